# baseline (speedup 1.0000x reference)
import functools

import jax
import jax.numpy as jnp
from jax import lax
from jax.experimental import pallas as pl
from jax.experimental.pallas import tpu as pltpu

D = 100
K = 8


def _tc_gather_sum(blk, rr, weight):
    B = blk.shape[0]
    nstep = B // K

    def body(blk_ref, rr_ref, *refs):
        wrefs = refs[:K]
        out_ref, acc_ref = refs[K], refs[K + 1]
        i = pl.program_id(0)

        @pl.when(i == 0)
        def _():
            acc_ref[...] = jnp.zeros_like(acc_ref)

        acc = acc_ref[...]
        rows = lax.broadcasted_iota(jnp.int32, (8, 1), 0)
        for j in range(K):
            sel = (rows == rr_ref[i * K + j]).astype(jnp.float32)
            acc += jnp.sum(wrefs[j][...] * sel, axis=0, keepdims=True)
        acc_ref[...] = acc

        @pl.when(i == nstep - 1)
        def _():
            out_ref[...] = acc_ref[...]

    grid_spec = pltpu.PrefetchScalarGridSpec(
        num_scalar_prefetch=2,
        grid=(nstep,),
        in_specs=[
            pl.BlockSpec(
                (8, D),
                functools.partial(
                    lambda j, i, blk_ref, rr_ref: (blk_ref[i * K + j], 0), j))
            for j in range(K)
        ],
        out_specs=pl.BlockSpec((1, D), lambda i, blk_ref, rr_ref: (0, 0)),
        scratch_shapes=[pltpu.VMEM((1, D), jnp.float32)],
    )
    return pl.pallas_call(
        body,
        grid_spec=grid_spec,
        out_shape=jax.ShapeDtypeStruct((1, D), jnp.float32),
    )(blk, rr, *([weight] * K))


def kernel(input, weight):
    idx = input.astype(jnp.int32)
    return _tc_gather_sum(idx // 8, idx % 8, weight)[0]


# speedup vs baseline: 2.0322x; 2.0322x over previous
import functools

import jax
import jax.numpy as jnp
from jax import lax
from jax.experimental import pallas as pl
from jax.experimental.pallas import tpu as pltpu
from jax.experimental.pallas import tpu_sc as plsc

D = 100
LANES = 16
COL_OFFS = (0, 16, 32, 48, 64, 80, 84)
NACC = len(COL_OFFS)
ACC_W = NACC * LANES

NC = 2
NS = 16
NW = NC * NS

GRP = 64


def _sc_embed_sum(input_idx, weight):
    B = input_idx.shape[0]
    BPW = B // NW
    NGRP = BPW // GRP

    mesh = plsc.VectorSubcoreMesh(core_axis_name="c", subcore_axis_name="s")

    @functools.partial(
        pl.kernel,
        out_type=jax.ShapeDtypeStruct((NW, ACC_W), jnp.float32),
        mesh=mesh,
        scratch_types=[
            pltpu.VMEM((BPW,), jnp.int32),
            pltpu.VMEM((GRP, D), jnp.float32),
            pltpu.VMEM((ACC_W,), jnp.float32),
            pltpu.SemaphoreType.DMA,
            pltpu.SemaphoreType.DMA,
            pltpu.SemaphoreType.DMA,
            pltpu.SemaphoreType.DMA,
        ],
    )
    def k(idx_hbm, tbl_hbm, out_hbm, idx_v, rows_v, acc_v, *sems):
        cid = lax.axis_index("c")
        sid = lax.axis_index("s")
        wid = sid * NC + cid
        base = wid * BPW

        pltpu.sync_copy(idx_hbm.at[pl.ds(base, BPW)], idx_v)

        def body(g, accs):
            v = idx_v[pl.ds(g * GRP, GRP)]
            for lane in range(GRP):
                pltpu.async_copy(tbl_hbm.at[v[lane]], rows_v.at[lane],
                                 sems[lane % 4])
            for q in range(4):
                pltpu.make_async_copy(
                    tbl_hbm.at[pl.ds(0, GRP // 4)],
                    rows_v.at[pl.ds(q * (GRP // 4), GRP // 4)],
                    sems[q]).wait()
            for lane in range(GRP):
                accs = tuple(
                    accs[i] + rows_v[lane, pl.ds(COL_OFFS[i], LANES)]
                    for i in range(NACC)
                )
            return accs

        zero = jnp.zeros((LANES,), jnp.float32)
        accs = lax.fori_loop(0, NGRP, body, (zero,) * NACC)

        for i in range(NACC):
            acc_v[pl.ds(i * LANES, LANES)] = accs[i]
        pltpu.sync_copy(acc_v, out_hbm.at[wid])

    return k(input_idx, weight)


TC_K = 8
TC_SHARE = 4096


def _tc_gather_sum(blk, rr, weight):
    nstep = blk.shape[0] // TC_K

    def body(blk_ref, rr_ref, *refs):
        wrefs = refs[:TC_K]
        out_ref, acc_ref = refs[TC_K], refs[TC_K + 1]
        i = pl.program_id(0)

        @pl.when(i == 0)
        def _():
            acc_ref[...] = jnp.zeros_like(acc_ref)

        acc = acc_ref[...]
        rows = lax.broadcasted_iota(jnp.int32, (8, 1), 0)
        for j in range(TC_K):
            sel = (rows == rr_ref[i * TC_K + j]).astype(jnp.float32)
            acc += jnp.sum(wrefs[j][...] * sel, axis=0, keepdims=True)
        acc_ref[...] = acc

        @pl.when(i == nstep - 1)
        def _():
            out_ref[...] = acc_ref[...]

    grid_spec = pltpu.PrefetchScalarGridSpec(
        num_scalar_prefetch=2,
        grid=(nstep,),
        in_specs=[
            pl.BlockSpec(
                (8, D),
                functools.partial(
                    lambda j, i, blk_ref, rr_ref: (blk_ref[i * TC_K + j], 0),
                    j))
            for j in range(TC_K)
        ],
        out_specs=pl.BlockSpec((1, D), lambda i, blk_ref, rr_ref: (0, 0)),
        scratch_shapes=[pltpu.VMEM((1, D), jnp.float32)],
    )
    return pl.pallas_call(
        body,
        grid_spec=grid_spec,
        out_shape=jax.ShapeDtypeStruct((1, D), jnp.float32),
    )(blk, rr, *([weight] * TC_K))


def kernel(input, weight):
    idx = input.astype(jnp.int32)
    idx_sc = idx[:-TC_SHARE]
    idx_tc = idx[-TC_SHARE:]
    part = _sc_embed_sum(idx_sc, weight)
    tc = _tc_gather_sum(idx_tc // 8, idx_tc % 8, weight)
    w = part.sum(axis=0)
    return jnp.concatenate([w[:96], w[108:112]]) + tc[0]
